# baseline (speedup 1.0000x reference)
import functools

import jax
import jax.numpy as jnp
from jax import lax
from jax.experimental import pallas as pl
from jax.experimental.pallas import tpu as pltpu
from jax.experimental.pallas import tpu_sc as plsc

DIM = 32
CHUNK = 128
NJ = 7813
NSUP = NJ * 32


JPW = (NJ + 31) // 32


def _detile_body(num_cores, wt_hbm, w2_hbm, tbuf, rbuf, rsem, wsem):
    wid = lax.axis_index("s") * num_cores + lax.axis_index("c")
    j0 = wid * JPW
    nj = jnp.minimum(JPW, NJ - j0)
    iota = lax.iota(jnp.int32, 16)
    zero = iota * 0

    def fire_read(j, par):
        pltpu.async_copy(wt_hbm.at[:, pl.ds(j * CHUNK, CHUNK)],
                         tbuf.at[par, :, pl.ds(0, CHUNK)], rsem[par])

    def wait_read(par):
        pltpu.make_async_copy(wt_hbm.at[:, pl.ds(0, CHUNK)],
                              tbuf.at[par, :, pl.ds(0, CHUNK)],
                              rsem[par]).wait()

    def wait_write(par):
        pltpu.make_async_copy(rbuf.at[par], w2_hbm.at[pl.ds(0, 32)],
                              wsem[par]).wait()

    def transpose(par):
        for su in range(32):
            for q in range(4):
                c = 32 * q + su
                for h in range(2):
                    v = plsc.load_gather(tbuf.at[par],
                                         [iota + 16 * h, zero + c])
                    rbuf[par, su, pl.ds(32 * q + 16 * h, 16)] = v

    def stage(jj, par):
        @pl.when(jj < nj)
        def _():
            wait_read(par)
            @pl.when(jj >= 2)
            def _():
                wait_write(par)
            transpose(par)
            pltpu.async_copy(rbuf.at[par],
                             w2_hbm.at[pl.ds(32 * (j0 + jj), 32)],
                             wsem[par])
            @pl.when(jj + 2 < nj)
            def _():
                fire_read(j0 + jj + 2, par)

    @pl.when(nj > 0)
    def _():
        fire_read(j0, 0)
    @pl.when(nj > 1)
    def _():
        fire_read(j0 + 1, 1)

    def body(i, carry):
        stage(2 * i, 0)
        stage(2 * i + 1, 1)
        return carry

    lax.fori_loop(0, (JPW + 1) // 2, body, jnp.int32(0), unroll=False)
    @pl.when(nj >= 1)
    def _():
        wait_write(0)
    @pl.when(nj >= 2)
    def _():
        wait_write(1)


def _detile_sc(wt, num_cores, mesh):
    k = pl.kernel(
        functools.partial(_detile_body, num_cores),
        out_type=jax.ShapeDtypeStruct((NSUP, CHUNK), jnp.float32),
        mesh=mesh,
        compiler_params=pltpu.CompilerParams(needs_layout_passes=False),
        scratch_types=[
            pltpu.VMEM((2, 32, 144), jnp.float32),
            pltpu.VMEM((2, 32, CHUNK), jnp.float32),
            [pltpu.SemaphoreType.DMA, pltpu.SemaphoreType.DMA],
            [pltpu.SemaphoreType.DMA, pltpu.SemaphoreType.DMA],
        ],
    )
    return k(wt)


def _gather_body(n_per_w, num_cores, idx_hbm, w2_hbm, out_hbm,
                 ibuf, ubuf, sbuf, obuf, sem, sem2):
    wid = lax.axis_index("s") * num_cores + lax.axis_index("c")
    pltpu.sync_copy(idx_hbm.at[wid], ibuf)

    iota = lax.iota(jnp.int32, 16)
    for cc in range(n_per_w):
        for g in range(8):
            r = ibuf[cc, pl.ds(16 * g, 16)]
            ubuf[cc, pl.ds(16 * g, 16)] = ((r >> 7) << 5) + (r & 31)

    copies = [None, None]
    copies[0] = pltpu.async_copy(w2_hbm.at[ubuf.at[0]], sbuf.at[0], sem)

    def extract_group(g, carry):
        cc, par = carry
        rvec = ibuf[cc, pl.ds(16 * g, 16)]
        scolv = ((rvec >> 5) & 3) << 5
        for l in range(16):
            scol = scolv[l]
            srow = 16 * g + l
            drow = 4 * g + (l >> 2)
            dcol = (l & 3) * 32
            for h in range(2):
                v = sbuf[par, srow, pl.ds(scol + 16 * h, 16)]
                obuf[par, drow, pl.ds(dcol + 16 * h, 16)] = v
        return carry

    out_copies = [None, None]
    for cc in range(n_per_w):
        par = cc % 2
        copies[par].wait()
        if cc + 1 < n_per_w:
            copies[(cc + 1) % 2] = pltpu.async_copy(
                w2_hbm.at[ubuf.at[cc + 1]], sbuf.at[(cc + 1) % 2], sem)
        if out_copies[par] is not None:
            out_copies[par].wait()
        lax.fori_loop(0, 8, extract_group, (jnp.int32(cc), jnp.int32(par)),
                      unroll=False)
        out_copies[par] = pltpu.async_copy(
            obuf.at[par], out_hbm.at[pl.ds((wid * n_per_w + cc) * 32, 32)],
            sem2)
    for oc in out_copies:
        if oc is not None:
            oc.wait()


def kernel(indices, weight):
    batch, n_fields = indices.shape
    total = batch * n_fields
    n_chunks = total // CHUNK

    info = plsc.get_sparse_core_info()
    num_workers = info.num_cores * info.num_subcores
    n_per_w = n_chunks // num_workers

    wt = weight.T
    mesh = plsc.VectorSubcoreMesh(core_axis_name="c", subcore_axis_name="s")
    w2 = _detile_sc(wt, info.num_cores, mesh)

    idx3 = indices.reshape(num_workers, n_per_w, CHUNK).astype(jnp.int32)
    k = pl.kernel(
        functools.partial(_gather_body, n_per_w, info.num_cores),
        out_type=jax.ShapeDtypeStruct((total // 4, CHUNK), jnp.float32),
        mesh=mesh,
        compiler_params=pltpu.CompilerParams(needs_layout_passes=False),
        scratch_types=[
            pltpu.VMEM((n_per_w, CHUNK), jnp.int32),
            pltpu.VMEM((n_per_w, CHUNK), jnp.int32),
            pltpu.VMEM((2, CHUNK, CHUNK), jnp.float32),
            pltpu.VMEM((2, 32, CHUNK), jnp.float32),
            pltpu.SemaphoreType.DMA,
            pltpu.SemaphoreType.DMA,
        ],
    )
    out = k(idx3, w2)
    return out.reshape(batch, n_fields, DIM)


# speedup vs baseline: 3.5228x; 3.5228x over previous
import functools

import jax
import jax.numpy as jnp
from jax import lax
from jax.experimental import pallas as pl
from jax.experimental.pallas import tpu as pltpu
from jax.experimental.pallas import tpu_sc as plsc

DIM = 32
CHUNK = 128
NJ = 7813
NSUP = NJ * 32


SUBJ = 64


def _transpose_body(wt_ref, w2_ref):
    for t4 in range(SUBJ // 4):
        x = wt_ref[:, pl.ds(t4 * 4 * CHUNK, 4 * CHUNK)]
        v = jnp.concatenate([x[:, 128 * k:128 * (k + 1)] for k in range(4)],
                            axis=0)
        y = v.T
        for k in range(4):
            w2_ref[pl.ds((t4 * 4 + k) * 32, 32), :] = jnp.concatenate(
                [y[32 * q:32 * (q + 1), 32 * k:32 * (k + 1)] for q in range(4)],
                axis=1)


def _detile_tc(wt):
    njb = (NJ + SUBJ - 1) // SUBJ
    return pl.pallas_call(
        _transpose_body,
        grid=(njb,),
        in_specs=[pl.BlockSpec((32, SUBJ * CHUNK), lambda j: (0, j))],
        out_specs=pl.BlockSpec((SUBJ * 32, CHUNK), lambda j: (j, 0)),
        out_shape=jax.ShapeDtypeStruct((njb * SUBJ * 32, CHUNK), jnp.float32),
    )(wt)


def _gather_body(n_per_w, num_cores, idx_hbm, w2_hbm, out_hbm,
                 ibuf, ubuf, sbuf, obuf, sem, sem2):
    wid = lax.axis_index("s") * num_cores + lax.axis_index("c")
    pltpu.sync_copy(idx_hbm.at[wid], ibuf)

    iota = lax.iota(jnp.int32, 16)
    for cc in range(n_per_w):
        for g in range(8):
            r = ibuf[cc, pl.ds(16 * g, 16)]
            ubuf[cc, pl.ds(16 * g, 16)] = ((r >> 7) << 5) + (r & 31)

    copies = [None, None]
    copies[0] = pltpu.async_copy(w2_hbm.at[ubuf.at[0]], sbuf.at[0], sem)

    def extract_group(g, carry):
        cc, par = carry
        rvec = ibuf[cc, pl.ds(16 * g, 16)]
        scolv = ((rvec >> 5) & 3) << 5
        for l in range(16):
            scol = scolv[l]
            srow = 16 * g + l
            drow = 4 * g + (l >> 2)
            dcol = (l & 3) * 32
            for h in range(2):
                v = sbuf[par, srow, pl.ds(scol + 16 * h, 16)]
                obuf[par, drow, pl.ds(dcol + 16 * h, 16)] = v
        return carry

    out_copies = [None, None]
    for cc in range(n_per_w):
        par = cc % 2
        copies[par].wait()
        if cc + 1 < n_per_w:
            copies[(cc + 1) % 2] = pltpu.async_copy(
                w2_hbm.at[ubuf.at[cc + 1]], sbuf.at[(cc + 1) % 2], sem)
        if out_copies[par] is not None:
            out_copies[par].wait()
        lax.fori_loop(0, 8, extract_group, (jnp.int32(cc), jnp.int32(par)),
                      unroll=False)
        out_copies[par] = pltpu.async_copy(
            obuf.at[par], out_hbm.at[pl.ds((wid * n_per_w + cc) * 32, 32)],
            sem2)
    for oc in out_copies:
        if oc is not None:
            oc.wait()


def kernel(indices, weight):
    batch, n_fields = indices.shape
    total = batch * n_fields
    n_chunks = total // CHUNK

    info = plsc.get_sparse_core_info()
    num_workers = info.num_cores * info.num_subcores
    n_per_w = n_chunks // num_workers

    wt = weight.T
    mesh = plsc.VectorSubcoreMesh(core_axis_name="c", subcore_axis_name="s")
    w2 = _detile_tc(wt)

    idx3 = indices.reshape(num_workers, n_per_w, CHUNK).astype(jnp.int32)
    k = pl.kernel(
        functools.partial(_gather_body, n_per_w, info.num_cores),
        out_type=jax.ShapeDtypeStruct((total // 4, CHUNK), jnp.float32),
        mesh=mesh,
        compiler_params=pltpu.CompilerParams(needs_layout_passes=False),
        scratch_types=[
            pltpu.VMEM((n_per_w, CHUNK), jnp.int32),
            pltpu.VMEM((n_per_w, CHUNK), jnp.int32),
            pltpu.VMEM((2, CHUNK, CHUNK), jnp.float32),
            pltpu.VMEM((2, 32, CHUNK), jnp.float32),
            pltpu.SemaphoreType.DMA,
            pltpu.SemaphoreType.DMA,
        ],
    )
    out = k(idx3, w2)
    return out.reshape(batch, n_fields, DIM)


# speedup vs baseline: 3.9322x; 1.1162x over previous
import functools

import jax
import jax.numpy as jnp
from jax import lax
from jax.experimental import pallas as pl
from jax.experimental.pallas import tpu as pltpu
from jax.experimental.pallas import tpu_sc as plsc

DIM = 32
CHUNK = 128
NJ = 7813
NSUP = NJ * 32


SUBJ = 128


def _transpose_body(wt_ref, w2_ref):
    for t4 in range(SUBJ // 4):
        x = wt_ref[:, pl.ds(t4 * 4 * CHUNK, 4 * CHUNK)]
        v = jnp.concatenate([x[:, 128 * k:128 * (k + 1)] for k in range(4)],
                            axis=0)
        y = v.T
        for k in range(4):
            w2_ref[pl.ds((t4 * 4 + k) * 32, 32), :] = jnp.concatenate(
                [y[32 * q:32 * (q + 1), 32 * k:32 * (k + 1)] for q in range(4)],
                axis=1)


def _detile_tc(wt):
    njb = (NJ + SUBJ - 1) // SUBJ
    return pl.pallas_call(
        _transpose_body,
        grid=(njb,),
        in_specs=[pl.BlockSpec((32, SUBJ * CHUNK), lambda j: (0, j))],
        out_specs=pl.BlockSpec((SUBJ * 32, CHUNK), lambda j: (j, 0)),
        out_shape=jax.ShapeDtypeStruct((njb * SUBJ * 32, CHUNK), jnp.float32),
    )(wt)


def _gather_body(n_per_w, num_cores, idx_hbm, w2_hbm, out_hbm,
                 ibuf, ubuf, sbuf, obuf, sem, sem2):
    wid = lax.axis_index("s") * num_cores + lax.axis_index("c")
    pltpu.sync_copy(idx_hbm.at[wid], ibuf)

    iota = lax.iota(jnp.int32, 16)
    for cc in range(n_per_w):
        for g in range(8):
            r = ibuf[cc, pl.ds(16 * g, 16)]
            ubuf[cc, pl.ds(16 * g, 16)] = ((r >> 7) << 5) + (r & 31)

    copies = [None, None]
    copies[0] = pltpu.async_copy(w2_hbm.at[ubuf.at[0]], sbuf.at[0], sem)

    def extract_group(g, carry):
        cc, par = carry
        rvec = ibuf[cc, pl.ds(16 * g, 16)]
        scolv = ((rvec >> 5) & 3) << 5
        for l in range(16):
            scol = scolv[l]
            srow = 16 * g + l
            drow = 4 * g + (l >> 2)
            dcol = (l & 3) * 32
            for h in range(2):
                v = sbuf[par, srow, pl.ds(scol + 16 * h, 16)]
                obuf[par, drow, pl.ds(dcol + 16 * h, 16)] = v
        return carry

    out_copies = [None, None]
    for cc in range(n_per_w):
        par = cc % 2
        copies[par].wait()
        if cc + 1 < n_per_w:
            copies[(cc + 1) % 2] = pltpu.async_copy(
                w2_hbm.at[ubuf.at[cc + 1]], sbuf.at[(cc + 1) % 2], sem)
        if out_copies[par] is not None:
            out_copies[par].wait()
        lax.fori_loop(0, 8, extract_group, (jnp.int32(cc), jnp.int32(par)),
                      unroll=False)
        out_copies[par] = pltpu.async_copy(
            obuf.at[par], out_hbm.at[pl.ds((wid * n_per_w + cc) * 32, 32)],
            sem2)
    for oc in out_copies:
        if oc is not None:
            oc.wait()


def kernel(indices, weight):
    batch, n_fields = indices.shape
    total = batch * n_fields
    n_chunks = total // CHUNK

    info = plsc.get_sparse_core_info()
    num_workers = info.num_cores * info.num_subcores
    n_per_w = n_chunks // num_workers

    wt = weight.T
    mesh = plsc.VectorSubcoreMesh(core_axis_name="c", subcore_axis_name="s")
    w2 = _detile_tc(wt)

    idx3 = indices.reshape(num_workers, n_per_w, CHUNK).astype(jnp.int32)
    k = pl.kernel(
        functools.partial(_gather_body, n_per_w, info.num_cores),
        out_type=jax.ShapeDtypeStruct((total // 4, CHUNK), jnp.float32),
        mesh=mesh,
        compiler_params=pltpu.CompilerParams(needs_layout_passes=False),
        scratch_types=[
            pltpu.VMEM((n_per_w, CHUNK), jnp.int32),
            pltpu.VMEM((n_per_w, CHUNK), jnp.int32),
            pltpu.VMEM((2, CHUNK, CHUNK), jnp.float32),
            pltpu.VMEM((2, 32, CHUNK), jnp.float32),
            pltpu.SemaphoreType.DMA,
            pltpu.SemaphoreType.DMA,
        ],
    )
    out = k(idx3, w2)
    return out.reshape(batch, n_fields, DIM)


# speedup vs baseline: 4.0435x; 1.0283x over previous
import functools

import jax
import jax.numpy as jnp
from jax import lax
from jax.experimental import pallas as pl
from jax.experimental.pallas import tpu as pltpu
from jax.experimental.pallas import tpu_sc as plsc

DIM = 32
CHUNK = 128
NJ = 7813
NSUP = NJ * 32


SUBJ = 256


def _transpose_body(wt_ref, w2_ref):
    for t4 in range(SUBJ // 4):
        x = wt_ref[:, pl.ds(t4 * 4 * CHUNK, 4 * CHUNK)]
        v = jnp.concatenate([x[:, 128 * k:128 * (k + 1)] for k in range(4)],
                            axis=0)
        y = v.T
        for k in range(4):
            w2_ref[pl.ds((t4 * 4 + k) * 32, 32), :] = jnp.concatenate(
                [y[32 * q:32 * (q + 1), 32 * k:32 * (k + 1)] for q in range(4)],
                axis=1)


def _detile_tc(wt):
    njb = (NJ + SUBJ - 1) // SUBJ
    return pl.pallas_call(
        _transpose_body,
        grid=(njb,),
        in_specs=[pl.BlockSpec((32, SUBJ * CHUNK), lambda j: (0, j))],
        out_specs=pl.BlockSpec((SUBJ * 32, CHUNK), lambda j: (j, 0)),
        out_shape=jax.ShapeDtypeStruct((njb * SUBJ * 32, CHUNK), jnp.float32),
    )(wt)


def _gather_body(n_per_w, num_cores, idx_hbm, w2_hbm, out_hbm,
                 ibuf, ubuf, sbuf, obuf, sem, sem2):
    wid = lax.axis_index("s") * num_cores + lax.axis_index("c")
    pltpu.sync_copy(idx_hbm.at[wid], ibuf)

    iota = lax.iota(jnp.int32, 16)
    for cc in range(n_per_w):
        for g in range(8):
            r = ibuf[cc, pl.ds(16 * g, 16)]
            ubuf[cc, pl.ds(16 * g, 16)] = ((r >> 7) << 5) + (r & 31)

    copies = [None, None]
    copies[0] = pltpu.async_copy(w2_hbm.at[ubuf.at[0]], sbuf.at[0], sem)

    def extract_group(g, carry):
        cc, par = carry
        rvec = ibuf[cc, pl.ds(16 * g, 16)]
        scolv = ((rvec >> 5) & 3) << 5
        for l in range(16):
            scol = scolv[l]
            srow = 16 * g + l
            drow = 4 * g + (l >> 2)
            dcol = (l & 3) * 32
            for h in range(2):
                v = sbuf[par, srow, pl.ds(scol + 16 * h, 16)]
                obuf[par, drow, pl.ds(dcol + 16 * h, 16)] = v
        return carry

    out_copies = [None, None]
    for cc in range(n_per_w):
        par = cc % 2
        copies[par].wait()
        if cc + 1 < n_per_w:
            copies[(cc + 1) % 2] = pltpu.async_copy(
                w2_hbm.at[ubuf.at[cc + 1]], sbuf.at[(cc + 1) % 2], sem)
        if out_copies[par] is not None:
            out_copies[par].wait()
        lax.fori_loop(0, 8, extract_group, (jnp.int32(cc), jnp.int32(par)),
                      unroll=False)
        out_copies[par] = pltpu.async_copy(
            obuf.at[par], out_hbm.at[pl.ds((wid * n_per_w + cc) * 32, 32)],
            sem2)
    for oc in out_copies:
        if oc is not None:
            oc.wait()


def kernel(indices, weight):
    batch, n_fields = indices.shape
    total = batch * n_fields
    n_chunks = total // CHUNK

    info = plsc.get_sparse_core_info()
    num_workers = info.num_cores * info.num_subcores
    n_per_w = n_chunks // num_workers

    wt = weight.T
    mesh = plsc.VectorSubcoreMesh(core_axis_name="c", subcore_axis_name="s")
    w2 = _detile_tc(wt)

    idx3 = indices.reshape(num_workers, n_per_w, CHUNK).astype(jnp.int32)
    k = pl.kernel(
        functools.partial(_gather_body, n_per_w, info.num_cores),
        out_type=jax.ShapeDtypeStruct((total // 4, CHUNK), jnp.float32),
        mesh=mesh,
        compiler_params=pltpu.CompilerParams(needs_layout_passes=False),
        scratch_types=[
            pltpu.VMEM((n_per_w, CHUNK), jnp.int32),
            pltpu.VMEM((n_per_w, CHUNK), jnp.int32),
            pltpu.VMEM((2, CHUNK, CHUNK), jnp.float32),
            pltpu.VMEM((2, 32, CHUNK), jnp.float32),
            pltpu.SemaphoreType.DMA,
            pltpu.SemaphoreType.DMA,
        ],
    )
    out = k(idx3, w2)
    return out.reshape(batch, n_fields, DIM)
